# baseline (speedup 1.0000x reference)
import functools

import jax
import jax.numpy as jnp
from jax import lax
from jax.experimental import pallas as pl
from jax.experimental.pallas import tpu as pltpu
from jax.experimental.pallas import tpu_sc as plsc

NC = 2
NS = 16
NW = NC * NS
G = 128


def _make_gather(n_g: int, D: int):
    mesh = plsc.VectorSubcoreMesh(core_axis_name="c", subcore_axis_name="s")

    @functools.partial(
        pl.kernel,
        mesh=mesh,
        out_type=jax.ShapeDtypeStruct((NW, n_g, G, D), jnp.float32),
        scratch_types=[
            pltpu.VMEM((n_g, G), jnp.int32),
            pltpu.VMEM((G, D), jnp.float32),
            pltpu.SemaphoreType.DMA,
        ],
        compiler_params=pltpu.CompilerParams(use_tc_tiling_on_sc=False),
    )
    def gather_kernel(keys_hbm, idx_hbm, out_hbm, idx_v, rows_v, sem):
        wid = lax.axis_index("s") * NC + lax.axis_index("c")
        pltpu.sync_copy(idx_hbm.at[wid], idx_v)

        def body(g, carry):
            pltpu.async_copy(keys_hbm.at[idx_v.at[g]], rows_v, sem).wait()
            pltpu.sync_copy(rows_v, out_hbm.at[wid, g])
            return carry

        lax.fori_loop(0, n_g, body, 0)

    return gather_kernel


def kernel(uids, keys):
    B, H = uids.shape
    V, D = keys.shape
    T = B * H
    assert T % (NW * G) == 0
    n_g = T // (NW * G)
    idx = uids.reshape(NW, n_g, G)
    out = _make_gather(n_g, D)(keys, idx)
    return out.reshape(B, H, D)


# speedup vs baseline: 1.0461x; 1.0461x over previous
import functools

import jax
import jax.numpy as jnp
from jax import lax
from jax.experimental import pallas as pl
from jax.experimental.pallas import tpu as pltpu
from jax.experimental.pallas import tpu_sc as plsc

NC = 2
NS = 16
NW = NC * NS
G = 128
NBUF = 5


def _make_gather(n_g: int, D: int):
    assert n_g % NBUF == 0
    n_outer = n_g // NBUF
    mesh = plsc.VectorSubcoreMesh(core_axis_name="c", subcore_axis_name="s")

    @functools.partial(
        pl.kernel,
        mesh=mesh,
        out_type=jax.ShapeDtypeStruct((NW, n_g, G, D), jnp.float32),
        scratch_types=(
            [pltpu.VMEM((n_g, G), jnp.int32)]
            + [pltpu.VMEM((G, D), jnp.float32) for _ in range(NBUF)]
            + [pltpu.SemaphoreType.DMA for _ in range(NBUF)]
        ),
        compiler_params=pltpu.CompilerParams(use_tc_tiling_on_sc=False),
    )
    def gather_kernel(keys_hbm, idx_hbm, out_hbm, idx_v, *bufs_and_sems):
        bufs = bufs_and_sems[:NBUF]
        gsems = bufs_and_sems[NBUF:]
        wid = lax.axis_index("s") * NC + lax.axis_index("c")
        pltpu.sync_copy(idx_hbm.at[wid], idx_v)

        for b in range(NBUF):
            pltpu.async_copy(keys_hbm.at[idx_v.at[b]], bufs[b], gsems[b])

        def outer(o, carry):
            for b in range(NBUF):
                g = o * NBUF + b
                pltpu.make_async_copy(
                    keys_hbm.at[idx_v.at[g]], bufs[b], gsems[b]
                ).wait()
                pltpu.sync_copy(bufs[b], out_hbm.at[wid, g])

                @pl.when(o < n_outer - 1)
                def _():
                    pltpu.async_copy(
                        keys_hbm.at[idx_v.at[g + NBUF]], bufs[b], gsems[b]
                    )

            return carry

        lax.fori_loop(0, n_outer, outer, 0)

    return gather_kernel


def kernel(uids, keys):
    B, H = uids.shape
    V, D = keys.shape
    T = B * H
    assert T % (NW * G) == 0
    n_g = T // (NW * G)
    idx = uids.reshape(NW, n_g, G)
    out = _make_gather(n_g, D)(keys, idx)
    return out.reshape(B, H, D)


# speedup vs baseline: 1.0484x; 1.0022x over previous
import functools

import jax
import jax.numpy as jnp
from jax import lax
from jax.experimental import pallas as pl
from jax.experimental.pallas import tpu as pltpu
from jax.experimental.pallas import tpu_sc as plsc

NC = 2
NS = 16
NW = NC * NS
NBUF = 8


def _make_gather(B: int, H: int, D: int):
    rows_per_w = B // NW
    assert rows_per_w % NBUF == 0
    n_outer = rows_per_w // NBUF
    mesh = plsc.VectorSubcoreMesh(core_axis_name="c", subcore_axis_name="s")

    @functools.partial(
        pl.kernel,
        mesh=mesh,
        out_type=jax.ShapeDtypeStruct((B, H, D), jnp.float32),
        scratch_types=(
            [pltpu.VMEM((rows_per_w, H), jnp.int32)]
            + [pltpu.VMEM((H, D), jnp.float32) for _ in range(NBUF)]
            + [pltpu.SemaphoreType.DMA for _ in range(NBUF)]
        ),
        compiler_params=pltpu.CompilerParams(use_tc_tiling_on_sc=False),
    )
    def gather_kernel(keys_hbm, uids_hbm, out_hbm, idx_v, *bufs_and_sems):
        bufs = bufs_and_sems[:NBUF]
        gsems = bufs_and_sems[NBUF:]
        wid = lax.axis_index("s") * NC + lax.axis_index("c")
        base = wid * rows_per_w
        pltpu.sync_copy(uids_hbm.at[pl.ds(base, rows_per_w)], idx_v)

        for b in range(NBUF):
            pltpu.async_copy(keys_hbm.at[idx_v.at[b]], bufs[b], gsems[b])

        def outer(o, carry):
            for b in range(NBUF):
                r = o * NBUF + b
                pltpu.make_async_copy(
                    keys_hbm.at[idx_v.at[r]], bufs[b], gsems[b]
                ).wait()
                pltpu.sync_copy(bufs[b], out_hbm.at[base + r])

                @pl.when(o < n_outer - 1)
                def _():
                    pltpu.async_copy(
                        keys_hbm.at[idx_v.at[r + NBUF]], bufs[b], gsems[b]
                    )

            return carry

        lax.fori_loop(0, n_outer, outer, 0)

    return gather_kernel


def kernel(uids, keys):
    B, H = uids.shape
    V, D = keys.shape
    assert B % NW == 0
    return _make_gather(B, H, D)(keys, uids)
